# Initial kernel scaffold; baseline (speedup 1.0000x reference)
import jax
import jax.numpy as jnp
from jax.experimental import pallas as pl


def kernel(z, weight):
    raise NotImplementedError("write your pallas kernel here")


# speedup vs baseline: 1.2835x; 1.2835x over previous
import jax
import jax.numpy as jnp
from jax.experimental import pallas as pl
from jax.experimental.pallas import tpu as pltpu

_N_EMBED = 8192
_CODE_DIM = 256
_BETA = 0.25
_ROWS = 8192
_TILE = 256
_NUM_TILES = _ROWS // _TILE


def _vq_kernel(z_ref, w_ref, zq_ref, idx_ref, enc_ref, loss_ref, perp_ref,
               counts_acc, loss_acc):
    i = pl.program_id(0)
    z = z_ref[...]
    w = w_ref[...]

    zsq = jnp.sum(z * z, axis=1, keepdims=True)
    wsq = jnp.sum(w * w, axis=1)[None, :]
    zw = jax.lax.dot_general(z, w, (((1,), (1,)), ((), ())),
                             preferred_element_type=jnp.float32)
    scores = zsq + wsq - 2.0 * zw

    idx = jnp.argmin(scores, axis=1).astype(jnp.int32)
    idx_ref[0, 0, :] = idx

    col = jax.lax.broadcasted_iota(jnp.int32, (_TILE, _N_EMBED), 1)
    onehot = (col == idx[:, None]).astype(jnp.float32)
    enc_ref[...] = onehot

    zq = jax.lax.dot_general(onehot, w, (((1,), (0,)), ((), ())),
                             preferred_element_type=jnp.float32)
    zq_ref[...] = zq

    part_counts = jnp.sum(onehot, axis=0, keepdims=True)
    diff = zq - z
    part_loss = jnp.sum(diff * diff)

    @pl.when(i == 0)
    def _init():
        counts_acc[...] = part_counts
        loss_acc[0, 0] = part_loss

    @pl.when(i > 0)
    def _accum():
        counts_acc[...] += part_counts
        loss_acc[0, 0] += part_loss

    @pl.when(i == _NUM_TILES - 1)
    def _finish():
        loss = _BETA * loss_acc[0, 0] / (_ROWS * _CODE_DIM)
        loss_ref[...] = jnp.reshape(loss, (1, 1))
        avg = counts_acc[...] / _ROWS
        ent = jnp.sum(avg * jnp.log(avg + 1e-10))
        perp_ref[...] = jnp.reshape(jnp.exp(-ent), (1, 1))


def kernel(z, weight):
    b, c, h, w = z.shape
    zt = jnp.transpose(z, (0, 2, 3, 1))
    z_flat = zt.reshape(-1, c)

    out_shapes = (
        jax.ShapeDtypeStruct((_ROWS, _CODE_DIM), jnp.float32),
        jax.ShapeDtypeStruct((_NUM_TILES, 1, _TILE), jnp.int32),
        jax.ShapeDtypeStruct((_ROWS, _N_EMBED), jnp.float32),
        jax.ShapeDtypeStruct((1, 1), jnp.float32),
        jax.ShapeDtypeStruct((1, 1), jnp.float32),
    )
    grid = (_NUM_TILES,)
    zq_flat, idx, encodings, loss, perp = pl.pallas_call(
        _vq_kernel,
        grid=grid,
        in_specs=[
            pl.BlockSpec((_TILE, _CODE_DIM), lambda i: (i, 0)),
            pl.BlockSpec((_N_EMBED, _CODE_DIM), lambda i: (0, 0)),
        ],
        out_specs=(
            pl.BlockSpec((_TILE, _CODE_DIM), lambda i: (i, 0)),
            pl.BlockSpec((1, 1, _TILE), lambda i: (i, 0, 0)),
            pl.BlockSpec((_TILE, _N_EMBED), lambda i: (i, 0)),
            pl.BlockSpec((1, 1), lambda i: (0, 0)),
            pl.BlockSpec((1, 1), lambda i: (0, 0)),
        ),
        out_shape=out_shapes,
        scratch_shapes=[
            pltpu.VMEM((1, _N_EMBED), jnp.float32),
            pltpu.SMEM((1, 1), jnp.float32),
        ],
    )(z_flat, weight)

    z_q_out = jnp.transpose(zq_flat.reshape(b, h, w, c), (0, 3, 1, 2))
    encoding_indices = idx.reshape(b, h, w)
    return (z_q_out, loss[0, 0], perp[0, 0], encodings,
            encoding_indices)
